# baseline (speedup 1.0000x reference)
import jax
import jax.numpy as jnp
from jax import lax
from jax.experimental import pallas as pl
from jax.experimental.pallas import tpu as pltpu
from jax.experimental.pallas import tpu_sc as plsc

N_NODES = 10000
K_ITERS = 3
ALPHA = 0.5
D_FEAT = 256
DH = 128
NC, NS = 2, 16
NPAD = 10240
ROWS_PER_TILE = NPAD // NS
ROW_CHUNK = 32
N_ROW_CHUNKS = ROWS_PER_TILE // ROW_CHUNK
E_EDGES = 160000
EPAD = 163840
EDGES_PER_TILE = EPAD // NS
ECHUNK = 128
N_ECHUNKS = EDGES_PER_TILE // ECHUNK

_F32 = jnp.float32
_I32 = jnp.int32


def _sc_body(xp, srcr, dstr, outp, hpr,
             agg, deg2d, sbuf, dbuf, sadjA, sadjB, dadjA, dadjB,
             gbufA, gbufB, abuf, hbuf, xbuf, zbuf, zb16, ones16, degbuf,
             dinv1d, semA, semB, semI, semSA, semSB):
    c = lax.axis_index("c")
    s = lax.axis_index("s")
    half = jnp.float32(0.5)
    coff = c * DH

    zeros_v = jnp.zeros((16,), _F32)
    ones_v = jnp.ones((16,), _F32)

    def fill_body(r, _):
        ones16[r, pl.ds(0, 16)] = ones_v
        return 0
    lax.fori_loop(0, ECHUNK, fill_body, 0)

    def fillz_body(r, _):
        for j in range(DH // 16):
            zbuf[r, pl.ds(j * 16, 16)] = zeros_v
        zb16[r, pl.ds(0, 16)] = zeros_v
        return 0
    lax.fori_loop(0, ROW_CHUNK, fillz_body, 0)

    def zero_both(ch, _):
        nbase = s * ROWS_PER_TILE + ch * ROW_CHUNK
        pltpu.sync_copy(zbuf, agg.at[pl.ds(nbase, ROW_CHUNK), :])
        pltpu.sync_copy(zb16, deg2d.at[pl.ds(nbase, ROW_CHUNK), :])
        return 0
    lax.fori_loop(0, N_ROW_CHUNKS, zero_both, 0)
    plsc.subcore_barrier()

    pltpu.sync_copy(dstr.at[pl.ds(s * EDGES_PER_TILE, ECHUNK)], dbuf)

    def deg_pair(t, _):
        off = s * EDGES_PER_TILE + 2 * t * ECHUNK
        pltpu.async_copy(dstr.at[pl.ds(off + ECHUNK, ECHUNK)], sbuf, semA)
        pltpu.sync_copy(ones16, deg2d.at[dbuf], add=True)

        @pl.when(t < N_ECHUNKS // 2 - 1)
        def _():
            pltpu.async_copy(
                dstr.at[pl.ds(off + 2 * ECHUNK, ECHUNK)], dbuf, semB)

        pltpu.make_async_copy(dstr.at[pl.ds(0, ECHUNK)], sbuf, semA).wait()
        pltpu.sync_copy(ones16, deg2d.at[sbuf], add=True)

        @pl.when(t < N_ECHUNKS // 2 - 1)
        def _():
            pltpu.make_async_copy(
                dstr.at[pl.ds(0, ECHUNK)], dbuf, semB).wait()
        return 0
    lax.fori_loop(0, N_ECHUNKS // 2, deg_pair, 0)
    plsc.subcore_barrier()

    def newton_chunk(ch, _):
        nbase = s * ROWS_PER_TILE + ch * ROW_CHUNK
        pltpu.sync_copy(deg2d.at[pl.ds(nbase, ROW_CHUNK), :], degbuf)

        iota_v = lax.iota(_I32, 16)
        zero_i = jnp.zeros((16,), _I32)
        for t in range(ROW_CHUNK // 16):
            rows = jnp.full((16,), t * 16, _I32) + iota_v
            d = plsc.load_gather(degbuf, [rows, zero_i]) + ones_v
            sq = jnp.full((16,), 512.0, _F32)
            for thresh, seed in ((65536.0, 256.0), (16384.0, 128.0),
                                 (4096.0, 64.0), (1024.0, 32.0),
                                 (256.0, 16.0), (64.0, 8.0),
                                 (16.0, 4.0), (4.0, 2.0)):
                sq = jnp.where(d < jnp.float32(thresh),
                               jnp.full((16,), seed, _F32), sq)
            for _it in range(5):
                sq = (sq + d / sq) * half
            out_rows = jnp.full((16,), ch * ROW_CHUNK + t * 16, _I32) + iota_v
            plsc.store_scatter(dinv1d, [out_rows], ones_v / sq)
        return 0
    lax.fori_loop(0, N_ROW_CHUNKS, newton_chunk, 0)

    def h0_chunk(ch, _):
        nbase = s * ROWS_PER_TILE + ch * ROW_CHUNK
        gbase = c * NPAD + nbase
        pltpu.sync_copy(xp.at[pl.ds(nbase, ROW_CHUNK), pl.ds(coff, DH)], xbuf)

        def h0_body(r, _):
            dv = plsc.load_gather(
                dinv1d, [jnp.full((16,), ch * ROW_CHUNK + r, _I32)])
            for j in range(DH // 16):
                hbuf[r, pl.ds(j * 16, 16)] = xbuf[r, pl.ds(j * 16, 16)] * dv
            return 0
        lax.fori_loop(0, ROW_CHUNK, h0_body, 0)
        pltpu.sync_copy(hbuf, hpr.at[pl.ds(gbase, ROW_CHUNK), :])
        return 0
    lax.fori_loop(0, N_ROW_CHUNKS, h0_chunk, 0)
    plsc.subcore_barrier()

    coff_v = jnp.full((16,), c * NPAD, _I32)

    H = ECHUNK // 2

    def load_group(g):
        off = s * EDGES_PER_TILE + g * ECHUNK
        pltpu.sync_copy(srcr.at[pl.ds(off, ECHUNK)], sbuf)
        pltpu.sync_copy(dstr.at[pl.ds(off, ECHUNK)], dbuf)

    def build_half(sadj, dadj, base):
        for i in range(H // 16):
            sadj[pl.ds(i * 16, 16)] = sbuf[pl.ds(base + i * 16, 16)] + coff_v
            dadj[pl.ds(i * 16, 16)] = dbuf[pl.ds(base + i * 16, 16)]

    def edge_loop():
        load_group(0)

        def edge_group(g, _):
            @pl.when(g > 0)
            def _():
                pltpu.make_async_copy(gbufA, agg.at[dadjA], semSA).wait()
            build_half(sadjA, dadjA, 0)
            pltpu.async_copy(hpr.at[sadjA], gbufA, semA)

            @pl.when(g > 0)
            def _():
                pltpu.make_async_copy(gbufB, agg.at[dadjB], semSB).wait()
            build_half(sadjB, dadjB, H)
            pltpu.async_copy(hpr.at[sadjB], gbufB, semB)

            @pl.when(g < N_ECHUNKS - 1)
            def _():
                off = s * EDGES_PER_TILE + (g + 1) * ECHUNK
                pltpu.async_copy(srcr.at[pl.ds(off, ECHUNK)], sbuf, semI)
                pltpu.async_copy(dstr.at[pl.ds(off, ECHUNK)], dbuf, semI)

            pltpu.make_async_copy(hpr.at[sadjA], gbufA, semA).wait()
            pltpu.async_copy(gbufA, agg.at[dadjA], semSA, add=True)
            pltpu.make_async_copy(hpr.at[sadjB], gbufB, semB).wait()
            pltpu.async_copy(gbufB, agg.at[dadjB], semSB, add=True)

            @pl.when(g < N_ECHUNKS - 1)
            def _():
                pltpu.make_async_copy(
                    srcr.at[pl.ds(0, ECHUNK)], sbuf, semI).wait()
                pltpu.make_async_copy(
                    dstr.at[pl.ds(0, ECHUNK)], dbuf, semI).wait()
            return 0
        lax.fori_loop(0, N_ECHUNKS, edge_group, 0)
        pltpu.make_async_copy(gbufA, agg.at[dadjA], semSA).wait()
        pltpu.make_async_copy(gbufB, agg.at[dadjB], semSB).wait()

    def upd_chunk(ch, last):
        nbase = s * ROWS_PER_TILE + ch * ROW_CHUNK
        gbase = c * NPAD + nbase
        cpa = pltpu.make_async_copy(agg.at[pl.ds(nbase, ROW_CHUNK), :],
                                    abuf, semA)
        cph = pltpu.make_async_copy(hpr.at[pl.ds(gbase, ROW_CHUNK), :],
                                    hbuf, semB)
        cpx = pltpu.make_async_copy(
            xp.at[pl.ds(nbase, ROW_CHUNK), pl.ds(coff, DH)], xbuf, semI)
        cpa.start()
        cph.start()
        cpx.start()
        cpa.wait()
        cph.wait()
        cpx.wait()

        def upd_body(r, _):
            dv = plsc.load_gather(
                dinv1d, [jnp.full((16,), ch * ROW_CHUNK + r, _I32)])
            if not last:
                av = half * dv * dv
                bv = half * dv
            else:
                av = half * dv
                bv = jnp.full((16,), half, _F32)
            for j in range(DH // 16):
                sj = abuf[r, pl.ds(j * 16, 16)] + hbuf[r, pl.ds(j * 16, 16)]
                o = av * sj + bv * xbuf[r, pl.ds(j * 16, 16)]
                if last:
                    o = jnp.maximum(o, jnp.float32(0.0))
                abuf[r, pl.ds(j * 16, 16)] = o
            return 0
        lax.fori_loop(0, ROW_CHUNK, upd_body, 0)

        if not last:
            pltpu.sync_copy(abuf, hpr.at[pl.ds(gbase, ROW_CHUNK), :])
            pltpu.sync_copy(zbuf, agg.at[pl.ds(nbase, ROW_CHUNK), :])
        else:
            pltpu.sync_copy(
                abuf, outp.at[pl.ds(nbase, ROW_CHUNK), pl.ds(coff, DH)])
        return 0

    def hop01(k, _):
        edge_loop()
        plsc.subcore_barrier()
        lax.fori_loop(0, N_ROW_CHUNKS, lambda i, cc: upd_chunk(i, False), 0)
        plsc.subcore_barrier()
        return 0
    lax.fori_loop(0, K_ITERS - 1, hop01, 0)

    edge_loop()
    plsc.subcore_barrier()
    lax.fori_loop(0, N_ROW_CHUNKS, lambda i, cc: upd_chunk(i, True), 0)


@jax.jit
def _sc_propagate(xp, srcr, dstr):
    mesh = plsc.VectorSubcoreMesh(core_axis_name="c", subcore_axis_name="s",
                                  num_cores=NC, num_subcores=NS)
    f = pl.kernel(
        _sc_body,
        out_type=(jax.ShapeDtypeStruct((NPAD, D_FEAT), _F32),
                  jax.ShapeDtypeStruct((NC * NPAD, DH), _F32)),
        mesh=mesh,
        scratch_types=[
            pltpu.MemorySpace.VMEM_SHARED((NPAD, DH), _F32),
            pltpu.MemorySpace.VMEM_SHARED((NPAD, 16), _F32),
            pltpu.VMEM((ECHUNK,), _I32),
            pltpu.VMEM((ECHUNK,), _I32),
            pltpu.VMEM((ECHUNK // 2,), _I32),
            pltpu.VMEM((ECHUNK // 2,), _I32),
            pltpu.VMEM((ECHUNK // 2,), _I32),
            pltpu.VMEM((ECHUNK // 2,), _I32),
            pltpu.VMEM((ECHUNK // 2, DH), _F32),
            pltpu.VMEM((ECHUNK // 2, DH), _F32),
            pltpu.VMEM((ROW_CHUNK, DH), _F32),
            pltpu.VMEM((ROW_CHUNK, DH), _F32),
            pltpu.VMEM((ROW_CHUNK, DH), _F32),
            pltpu.VMEM((ROW_CHUNK, DH), _F32),
            pltpu.VMEM((ROW_CHUNK, 16), _F32),
            pltpu.VMEM((ECHUNK, 16), _F32),
            pltpu.VMEM((ROW_CHUNK, 16), _F32),
            pltpu.VMEM((ROWS_PER_TILE,), _F32),
            pltpu.SemaphoreType.DMA,
            pltpu.SemaphoreType.DMA,
            pltpu.SemaphoreType.DMA,
            pltpu.SemaphoreType.DMA,
            pltpu.SemaphoreType.DMA,
        ],
        compiler_params=pltpu.CompilerParams(use_tc_tiling_on_sc=False,
                                             needs_layout_passes=False),
        name="appnp_sc_propagate",
    )
    return f(xp, srcr, dstr)


def _mlp_body(p_ref, w1_ref, b1_ref, w2_ref, b2_ref, emb_ref, log_ref):
    pb = p_ref[...]
    emb = lax.dot_general(pb, w1_ref[...], (((1,), (1,)), ((), ())),
                          preferred_element_type=_F32) + b1_ref[...]
    emb_ref[...] = emb
    r = jnp.maximum(emb, jnp.float32(0.0))
    log_ref[...] = lax.dot_general(r, w2_ref[...], (((1,), (1,)), ((), ())),
                                   preferred_element_type=_F32) + b2_ref[...]


def _mlp(p, W1, b1, W2, b2):
    BR = 1000
    grid = (N_NODES // BR,)
    return pl.pallas_call(
        _mlp_body,
        grid=grid,
        in_specs=[
            pl.BlockSpec((BR, D_FEAT), lambda i: (i, 0)),
            pl.BlockSpec((D_FEAT, D_FEAT), lambda i: (0, 0)),
            pl.BlockSpec((1, D_FEAT), lambda i: (0, 0)),
            pl.BlockSpec((40, D_FEAT), lambda i: (0, 0)),
            pl.BlockSpec((1, 40), lambda i: (0, 0)),
        ],
        out_specs=[
            pl.BlockSpec((BR, D_FEAT), lambda i: (i, 0)),
            pl.BlockSpec((BR, 40), lambda i: (i, 0)),
        ],
        out_shape=[
            jax.ShapeDtypeStruct((N_NODES, D_FEAT), _F32),
            jax.ShapeDtypeStruct((N_NODES, 40), _F32),
        ],
    )(p, W1, b1, W2, b2)


def kernel(x, edge_index, W1, b1, W2, b2):
    src = edge_index[0].astype(_I32)
    dst = edge_index[1].astype(_I32)
    npad_e = EPAD - E_EDGES
    pad_idx = (N_NODES + jnp.arange(npad_e, dtype=_I32) % (NPAD - N_NODES))
    srcr = jnp.concatenate([src, pad_idx])
    dstr = jnp.concatenate([dst, pad_idx])
    xp = jnp.pad(x, ((0, NPAD - N_NODES), (0, 0)))
    outp, _ = _sc_propagate(xp, srcr, dstr)
    return _mlp(outp, W1, b1.reshape(1, -1), W2, b2.reshape(1, -1))


# speedup vs baseline: 1.1716x; 1.1716x over previous
import jax
import jax.numpy as jnp
from jax import lax
from jax.experimental import pallas as pl
from jax.experimental.pallas import tpu as pltpu
from jax.experimental.pallas import tpu_sc as plsc

N_NODES = 10000
K_ITERS = 3
ALPHA = 0.5
D_FEAT = 256
DH = 128
NC, NS = 2, 16
NPAD = 10240
ROWS_PER_TILE = NPAD // NS
ROW_CHUNK = 32
N_ROW_CHUNKS = ROWS_PER_TILE // ROW_CHUNK
E_EDGES = 160000
EPAD = 163840
EDGES_PER_TILE = EPAD // NS
ECHUNK = 128
N_ECHUNKS = EDGES_PER_TILE // ECHUNK

_F32 = jnp.float32
_I32 = jnp.int32


def _sc_body(xp, srcr, dstr, outp, hpr,
             agg, deg2d, sbuf, dbuf, sadjA, sadjB, dadjA, dadjB,
             gbufA, gbufB, abuf, hbuf, xbuf, zbuf, zb16, ones16, degbuf,
             dinv1d, semA, semB, semI):
    c = lax.axis_index("c")
    s = lax.axis_index("s")
    half = jnp.float32(0.5)
    coff = c * DH

    zeros_v = jnp.zeros((16,), _F32)
    ones_v = jnp.ones((16,), _F32)

    def fill_body(r, _):
        ones16[r, pl.ds(0, 16)] = ones_v
        return 0
    lax.fori_loop(0, ECHUNK, fill_body, 0)

    def fillz_body(r, _):
        for j in range(DH // 16):
            zbuf[r, pl.ds(j * 16, 16)] = zeros_v
        zb16[r, pl.ds(0, 16)] = zeros_v
        return 0
    lax.fori_loop(0, ROW_CHUNK, fillz_body, 0)

    def zero_both(ch, _):
        nbase = s * ROWS_PER_TILE + ch * ROW_CHUNK
        pltpu.sync_copy(zbuf, agg.at[pl.ds(nbase, ROW_CHUNK), :])
        pltpu.sync_copy(zb16, deg2d.at[pl.ds(nbase, ROW_CHUNK), :])
        return 0
    lax.fori_loop(0, N_ROW_CHUNKS, zero_both, 0)
    plsc.subcore_barrier()

    pltpu.sync_copy(dstr.at[pl.ds(s * EDGES_PER_TILE, ECHUNK)], dbuf)

    def deg_pair(t, _):
        off = s * EDGES_PER_TILE + 2 * t * ECHUNK
        pltpu.async_copy(dstr.at[pl.ds(off + ECHUNK, ECHUNK)], sbuf, semA)
        pltpu.sync_copy(ones16, deg2d.at[dbuf], add=True)

        @pl.when(t < N_ECHUNKS // 2 - 1)
        def _():
            pltpu.async_copy(
                dstr.at[pl.ds(off + 2 * ECHUNK, ECHUNK)], dbuf, semB)

        pltpu.make_async_copy(dstr.at[pl.ds(0, ECHUNK)], sbuf, semA).wait()
        pltpu.sync_copy(ones16, deg2d.at[sbuf], add=True)

        @pl.when(t < N_ECHUNKS // 2 - 1)
        def _():
            pltpu.make_async_copy(
                dstr.at[pl.ds(0, ECHUNK)], dbuf, semB).wait()
        return 0
    lax.fori_loop(0, N_ECHUNKS // 2, deg_pair, 0)
    plsc.subcore_barrier()

    def newton_chunk(ch, _):
        nbase = s * ROWS_PER_TILE + ch * ROW_CHUNK
        pltpu.sync_copy(deg2d.at[pl.ds(nbase, ROW_CHUNK), :], degbuf)

        iota_v = lax.iota(_I32, 16)
        zero_i = jnp.zeros((16,), _I32)
        for t in range(ROW_CHUNK // 16):
            rows = jnp.full((16,), t * 16, _I32) + iota_v
            d = plsc.load_gather(degbuf, [rows, zero_i]) + ones_v
            sq = jnp.full((16,), 512.0, _F32)
            for thresh, seed in ((65536.0, 256.0), (16384.0, 128.0),
                                 (4096.0, 64.0), (1024.0, 32.0),
                                 (256.0, 16.0), (64.0, 8.0),
                                 (16.0, 4.0), (4.0, 2.0)):
                sq = jnp.where(d < jnp.float32(thresh),
                               jnp.full((16,), seed, _F32), sq)
            for _it in range(5):
                sq = (sq + d / sq) * half
            out_rows = jnp.full((16,), ch * ROW_CHUNK + t * 16, _I32) + iota_v
            plsc.store_scatter(dinv1d, [out_rows], ones_v / sq)
        return 0
    lax.fori_loop(0, N_ROW_CHUNKS, newton_chunk, 0)

    def h0_chunk(ch, _):
        nbase = s * ROWS_PER_TILE + ch * ROW_CHUNK
        gbase = c * NPAD + nbase
        pltpu.sync_copy(xp.at[pl.ds(nbase, ROW_CHUNK), pl.ds(coff, DH)], xbuf)

        def h0_body(r, _):
            dv = plsc.load_gather(
                dinv1d, [jnp.full((16,), ch * ROW_CHUNK + r, _I32)])
            for j in range(DH // 16):
                hbuf[r, pl.ds(j * 16, 16)] = xbuf[r, pl.ds(j * 16, 16)] * dv
            return 0
        lax.fori_loop(0, ROW_CHUNK, h0_body, 0)
        pltpu.sync_copy(hbuf, hpr.at[pl.ds(gbase, ROW_CHUNK), :])
        return 0
    lax.fori_loop(0, N_ROW_CHUNKS, h0_chunk, 0)
    plsc.subcore_barrier()

    coff_v = jnp.full((16,), c * NPAD, _I32)

    H = ECHUNK // 2

    def load_group(g):
        off = s * EDGES_PER_TILE + g * ECHUNK
        pltpu.sync_copy(srcr.at[pl.ds(off, ECHUNK)], sbuf)
        pltpu.sync_copy(dstr.at[pl.ds(off, ECHUNK)], dbuf)

    def build_half(sadj, dadj, base):
        for i in range(H // 16):
            sadj[pl.ds(i * 16, 16)] = sbuf[pl.ds(base + i * 16, 16)] + coff_v
            dadj[pl.ds(i * 16, 16)] = dbuf[pl.ds(base + i * 16, 16)]

    def edge_loop():
        load_group(0)
        build_half(sadjA, dadjA, 0)
        build_half(sadjB, dadjB, H)
        gA = pltpu.async_copy(hpr.at[sadjA], gbufA, semA)

        def edge_group(g, _):
            pltpu.async_copy(hpr.at[sadjB], gbufB, semB)

            @pl.when(g < N_ECHUNKS - 1)
            def _():
                off = s * EDGES_PER_TILE + (g + 1) * ECHUNK
                pltpu.async_copy(srcr.at[pl.ds(off, ECHUNK)], sbuf, semI)
                pltpu.async_copy(dstr.at[pl.ds(off, ECHUNK)], dbuf, semI)

            pltpu.make_async_copy(hpr.at[sadjA], gbufA, semA).wait()
            pltpu.sync_copy(gbufA, agg.at[dadjA], add=True)

            @pl.when(g < N_ECHUNKS - 1)
            def _():
                pltpu.make_async_copy(
                    srcr.at[pl.ds(0, ECHUNK)], sbuf, semI).wait()
                pltpu.make_async_copy(
                    dstr.at[pl.ds(0, ECHUNK)], dbuf, semI).wait()
                build_half(sadjA, dadjA, 0)
                pltpu.async_copy(hpr.at[sadjA], gbufA, semA)

            pltpu.make_async_copy(hpr.at[sadjB], gbufB, semB).wait()
            pltpu.sync_copy(gbufB, agg.at[dadjB], add=True)

            @pl.when(g < N_ECHUNKS - 1)
            def _():
                build_half(sadjB, dadjB, H)
            return 0
        lax.fori_loop(0, N_ECHUNKS, edge_group, 0)

    def upd_chunk(ch, last):
        nbase = s * ROWS_PER_TILE + ch * ROW_CHUNK
        gbase = c * NPAD + nbase
        cpa = pltpu.make_async_copy(agg.at[pl.ds(nbase, ROW_CHUNK), :],
                                    abuf, semA)
        cph = pltpu.make_async_copy(hpr.at[pl.ds(gbase, ROW_CHUNK), :],
                                    hbuf, semB)
        cpx = pltpu.make_async_copy(
            xp.at[pl.ds(nbase, ROW_CHUNK), pl.ds(coff, DH)], xbuf, semI)
        cpa.start()
        cph.start()
        cpx.start()
        cpa.wait()
        cph.wait()
        cpx.wait()

        def upd_body(r, _):
            dv = plsc.load_gather(
                dinv1d, [jnp.full((16,), ch * ROW_CHUNK + r, _I32)])
            if not last:
                av = half * dv * dv
                bv = half * dv
            else:
                av = half * dv
                bv = jnp.full((16,), half, _F32)
            for j in range(DH // 16):
                sj = abuf[r, pl.ds(j * 16, 16)] + hbuf[r, pl.ds(j * 16, 16)]
                o = av * sj + bv * xbuf[r, pl.ds(j * 16, 16)]
                if last:
                    o = jnp.maximum(o, jnp.float32(0.0))
                abuf[r, pl.ds(j * 16, 16)] = o
            return 0
        lax.fori_loop(0, ROW_CHUNK, upd_body, 0)

        if not last:
            pltpu.sync_copy(abuf, hpr.at[pl.ds(gbase, ROW_CHUNK), :])
            pltpu.sync_copy(zbuf, agg.at[pl.ds(nbase, ROW_CHUNK), :])
        else:
            pltpu.sync_copy(
                abuf, outp.at[pl.ds(nbase, ROW_CHUNK), pl.ds(coff, DH)])
        return 0

    def hop01(k, _):
        edge_loop()
        plsc.subcore_barrier()
        lax.fori_loop(0, N_ROW_CHUNKS, lambda i, cc: upd_chunk(i, False), 0)
        plsc.subcore_barrier()
        return 0
    lax.fori_loop(0, K_ITERS - 1, hop01, 0)

    edge_loop()
    plsc.subcore_barrier()
    lax.fori_loop(0, N_ROW_CHUNKS, lambda i, cc: upd_chunk(i, True), 0)


@jax.jit
def _sc_propagate(xp, srcr, dstr):
    mesh = plsc.VectorSubcoreMesh(core_axis_name="c", subcore_axis_name="s",
                                  num_cores=NC, num_subcores=NS)
    f = pl.kernel(
        _sc_body,
        out_type=(jax.ShapeDtypeStruct((NPAD, D_FEAT), _F32),
                  jax.ShapeDtypeStruct((NC * NPAD, DH), _F32)),
        mesh=mesh,
        scratch_types=[
            pltpu.MemorySpace.VMEM_SHARED((NPAD, DH), _F32),
            pltpu.MemorySpace.VMEM_SHARED((NPAD, 16), _F32),
            pltpu.VMEM((ECHUNK,), _I32),
            pltpu.VMEM((ECHUNK,), _I32),
            pltpu.VMEM((ECHUNK // 2,), _I32),
            pltpu.VMEM((ECHUNK // 2,), _I32),
            pltpu.VMEM((ECHUNK // 2,), _I32),
            pltpu.VMEM((ECHUNK // 2,), _I32),
            pltpu.VMEM((ECHUNK // 2, DH), _F32),
            pltpu.VMEM((ECHUNK // 2, DH), _F32),
            pltpu.VMEM((ROW_CHUNK, DH), _F32),
            pltpu.VMEM((ROW_CHUNK, DH), _F32),
            pltpu.VMEM((ROW_CHUNK, DH), _F32),
            pltpu.VMEM((ROW_CHUNK, DH), _F32),
            pltpu.VMEM((ROW_CHUNK, 16), _F32),
            pltpu.VMEM((ECHUNK, 16), _F32),
            pltpu.VMEM((ROW_CHUNK, 16), _F32),
            pltpu.VMEM((ROWS_PER_TILE,), _F32),
            pltpu.SemaphoreType.DMA,
            pltpu.SemaphoreType.DMA,
            pltpu.SemaphoreType.DMA,
        ],
        compiler_params=pltpu.CompilerParams(use_tc_tiling_on_sc=False,
                                             needs_layout_passes=False),
        name="appnp_sc_propagate",
    )
    return f(xp, srcr, dstr)


def _mlp_body(p_ref, w1_ref, b1_ref, w2_ref, b2_ref, emb_ref, log_ref):
    pb = p_ref[...]
    emb = lax.dot_general(pb, w1_ref[...], (((1,), (1,)), ((), ())),
                          preferred_element_type=_F32) + b1_ref[...]
    emb_ref[...] = emb
    r = jnp.maximum(emb, jnp.float32(0.0))
    log_ref[...] = lax.dot_general(r, w2_ref[...], (((1,), (1,)), ((), ())),
                                   preferred_element_type=_F32) + b2_ref[...]


def _mlp(p, W1, b1, W2, b2):
    BR = 1000
    grid = (N_NODES // BR,)
    return pl.pallas_call(
        _mlp_body,
        grid=grid,
        in_specs=[
            pl.BlockSpec((BR, D_FEAT), lambda i: (i, 0)),
            pl.BlockSpec((D_FEAT, D_FEAT), lambda i: (0, 0)),
            pl.BlockSpec((1, D_FEAT), lambda i: (0, 0)),
            pl.BlockSpec((40, D_FEAT), lambda i: (0, 0)),
            pl.BlockSpec((1, 40), lambda i: (0, 0)),
        ],
        out_specs=[
            pl.BlockSpec((BR, D_FEAT), lambda i: (i, 0)),
            pl.BlockSpec((BR, 40), lambda i: (i, 0)),
        ],
        out_shape=[
            jax.ShapeDtypeStruct((N_NODES, D_FEAT), _F32),
            jax.ShapeDtypeStruct((N_NODES, 40), _F32),
        ],
    )(p, W1, b1, W2, b2)


def kernel(x, edge_index, W1, b1, W2, b2):
    src = edge_index[0].astype(_I32)
    dst = edge_index[1].astype(_I32)
    npad_e = EPAD - E_EDGES
    pad_idx = (N_NODES + jnp.arange(npad_e, dtype=_I32) % (NPAD - N_NODES))
    srcr = jnp.concatenate([src, pad_idx])
    dstr = jnp.concatenate([dst, pad_idx])
    xp = jnp.pad(x, ((0, NPAD - N_NODES), (0, 0)))
    outp, _ = _sc_propagate(xp, srcr, dstr)
    return _mlp(outp, W1, b1.reshape(1, -1), W2, b2.reshape(1, -1))


# speedup vs baseline: 1.1821x; 1.0090x over previous
import jax
import jax.numpy as jnp
from jax import lax
from jax.experimental import pallas as pl
from jax.experimental.pallas import tpu as pltpu
from jax.experimental.pallas import tpu_sc as plsc

N_NODES = 10000
K_ITERS = 3
ALPHA = 0.5
D_FEAT = 256
DH = 128
NC, NS = 2, 16
NPAD = 10240
ROWS_PER_TILE = NPAD // NS
ROW_CHUNK = 32
N_ROW_CHUNKS = ROWS_PER_TILE // ROW_CHUNK
E_EDGES = 160000
EPAD = 163840
EDGES_PER_TILE = EPAD // NS
ECHUNK = 128
N_ECHUNKS = EDGES_PER_TILE // ECHUNK

_F32 = jnp.float32
_I32 = jnp.int32


def _sc_body(xp, srcr, dstr, outp, hpr,
             agg, deg2d, sbuf, dbuf, sadjA, sadjB, dadjA, dadjB,
             gbufA, gbufB, abuf, hbuf, xbuf, zbuf, zb16, ones16, degbuf,
             dinv1d, semA, semB, semI):
    c = lax.axis_index("c")
    s = lax.axis_index("s")
    half = jnp.float32(0.5)
    coff = c * DH

    zeros_v = jnp.zeros((16,), _F32)
    ones_v = jnp.ones((16,), _F32)

    def fill_body(r, _):
        ones16[r, pl.ds(0, 16)] = ones_v
        return 0
    lax.fori_loop(0, ECHUNK, fill_body, 0)

    def fillz_body(r, _):
        for j in range(DH // 16):
            zbuf[r, pl.ds(j * 16, 16)] = zeros_v
        zb16[r, pl.ds(0, 16)] = zeros_v
        return 0
    lax.fori_loop(0, ROW_CHUNK, fillz_body, 0)

    def zero_both(ch, _):
        nbase = s * ROWS_PER_TILE + ch * ROW_CHUNK
        pltpu.sync_copy(zbuf, agg.at[pl.ds(nbase, ROW_CHUNK), :])
        pltpu.sync_copy(zb16, deg2d.at[pl.ds(nbase, ROW_CHUNK), :])
        return 0
    lax.fori_loop(0, N_ROW_CHUNKS, zero_both, 0)
    plsc.subcore_barrier()

    pltpu.sync_copy(dstr.at[pl.ds(s * EDGES_PER_TILE, ECHUNK)], dbuf)

    def deg_pair(t, _):
        off = s * EDGES_PER_TILE + 2 * t * ECHUNK
        pltpu.async_copy(dstr.at[pl.ds(off + ECHUNK, ECHUNK)], sbuf, semA)
        pltpu.sync_copy(ones16, deg2d.at[dbuf], add=True)

        @pl.when(t < N_ECHUNKS // 2 - 1)
        def _():
            pltpu.async_copy(
                dstr.at[pl.ds(off + 2 * ECHUNK, ECHUNK)], dbuf, semB)

        pltpu.make_async_copy(dstr.at[pl.ds(0, ECHUNK)], sbuf, semA).wait()
        pltpu.sync_copy(ones16, deg2d.at[sbuf], add=True)

        @pl.when(t < N_ECHUNKS // 2 - 1)
        def _():
            pltpu.make_async_copy(
                dstr.at[pl.ds(0, ECHUNK)], dbuf, semB).wait()
        return 0
    lax.fori_loop(0, N_ECHUNKS // 2, deg_pair, 0)
    plsc.subcore_barrier()

    def newton_chunk(ch, _):
        nbase = s * ROWS_PER_TILE + ch * ROW_CHUNK
        pltpu.sync_copy(deg2d.at[pl.ds(nbase, ROW_CHUNK), :], degbuf)

        iota_v = lax.iota(_I32, 16)
        zero_i = jnp.zeros((16,), _I32)
        for t in range(ROW_CHUNK // 16):
            rows = jnp.full((16,), t * 16, _I32) + iota_v
            d = plsc.load_gather(degbuf, [rows, zero_i]) + ones_v
            sq = jnp.full((16,), 512.0, _F32)
            for thresh, seed in ((65536.0, 256.0), (16384.0, 128.0),
                                 (4096.0, 64.0), (1024.0, 32.0),
                                 (256.0, 16.0), (64.0, 8.0),
                                 (16.0, 4.0), (4.0, 2.0)):
                sq = jnp.where(d < jnp.float32(thresh),
                               jnp.full((16,), seed, _F32), sq)
            for _it in range(5):
                sq = (sq + d / sq) * half
            out_rows = jnp.full((16,), ch * ROW_CHUNK + t * 16, _I32) + iota_v
            plsc.store_scatter(dinv1d, [out_rows], ones_v / sq)
        return 0
    lax.fori_loop(0, N_ROW_CHUNKS, newton_chunk, 0)

    HC = ROW_CHUNK // 2

    def h0_chunk(ch, _):
        nbase = s * ROWS_PER_TILE + ch * ROW_CHUNK
        gbase = c * NPAD + nbase

        @pl.when(nbase + ROW_CHUNK <= N_NODES)
        def _():
            pltpu.sync_copy(
                xp.at[pl.ds(nbase, ROW_CHUNK), pl.ds(coff, DH)], xbuf)

        @pl.when(nbase + ROW_CHUNK == N_NODES + HC)
        def _():
            pltpu.sync_copy(xp.at[pl.ds(nbase, HC), pl.ds(coff, DH)],
                            xbuf.at[pl.ds(0, HC), :])

        @pl.when(nbase < N_NODES)
        def _():
            def h0_body(r, _):
                dv = plsc.load_gather(
                    dinv1d, [jnp.full((16,), ch * ROW_CHUNK + r, _I32)])
                for j in range(DH // 16):
                    hbuf[r, pl.ds(j * 16, 16)] = (
                        xbuf[r, pl.ds(j * 16, 16)] * dv)
                return 0
            lax.fori_loop(0, ROW_CHUNK, h0_body, 0)
            pltpu.sync_copy(hbuf, hpr.at[pl.ds(gbase, ROW_CHUNK), :])
        return 0
    lax.fori_loop(0, N_ROW_CHUNKS, h0_chunk, 0)
    plsc.subcore_barrier()

    coff_v = jnp.full((16,), c * NPAD, _I32)

    H = ECHUNK // 2

    def load_group(g):
        off = s * EDGES_PER_TILE + g * ECHUNK
        pltpu.sync_copy(srcr.at[pl.ds(off, ECHUNK)], sbuf)
        pltpu.sync_copy(dstr.at[pl.ds(off, ECHUNK)], dbuf)

    def build_half(sadj, dadj, base):
        for i in range(H // 16):
            sadj[pl.ds(i * 16, 16)] = sbuf[pl.ds(base + i * 16, 16)] + coff_v
            dadj[pl.ds(i * 16, 16)] = dbuf[pl.ds(base + i * 16, 16)]

    def edge_loop():
        load_group(0)
        build_half(sadjA, dadjA, 0)
        build_half(sadjB, dadjB, H)
        gA = pltpu.async_copy(hpr.at[sadjA], gbufA, semA)

        def edge_group(g, _):
            pltpu.async_copy(hpr.at[sadjB], gbufB, semB)

            @pl.when(g < N_ECHUNKS - 1)
            def _():
                off = s * EDGES_PER_TILE + (g + 1) * ECHUNK
                pltpu.async_copy(srcr.at[pl.ds(off, ECHUNK)], sbuf, semI)
                pltpu.async_copy(dstr.at[pl.ds(off, ECHUNK)], dbuf, semI)

            pltpu.make_async_copy(hpr.at[sadjA], gbufA, semA).wait()
            pltpu.sync_copy(gbufA, agg.at[dadjA], add=True)

            @pl.when(g < N_ECHUNKS - 1)
            def _():
                pltpu.make_async_copy(
                    srcr.at[pl.ds(0, ECHUNK)], sbuf, semI).wait()
                pltpu.make_async_copy(
                    dstr.at[pl.ds(0, ECHUNK)], dbuf, semI).wait()
                build_half(sadjA, dadjA, 0)
                pltpu.async_copy(hpr.at[sadjA], gbufA, semA)

            pltpu.make_async_copy(hpr.at[sadjB], gbufB, semB).wait()
            pltpu.sync_copy(gbufB, agg.at[dadjB], add=True)

            @pl.when(g < N_ECHUNKS - 1)
            def _():
                build_half(sadjB, dadjB, H)
            return 0
        lax.fori_loop(0, N_ECHUNKS, edge_group, 0)

    def upd_chunk(ch, last):
        nbase = s * ROWS_PER_TILE + ch * ROW_CHUNK
        gbase = c * NPAD + nbase
        cpa = pltpu.make_async_copy(agg.at[pl.ds(nbase, ROW_CHUNK), :],
                                    abuf, semA)
        cph = pltpu.make_async_copy(hpr.at[pl.ds(gbase, ROW_CHUNK), :],
                                    hbuf, semB)
        cpx = pltpu.make_async_copy(
            xp.at[pl.ds(nbase, ROW_CHUNK), pl.ds(coff, DH)], xbuf, semI)
        cpxh = pltpu.make_async_copy(
            xp.at[pl.ds(nbase, HC), pl.ds(coff, DH)],
            xbuf.at[pl.ds(0, HC), :], semI)
        cpa.start()
        cph.start()

        @pl.when(nbase + ROW_CHUNK <= N_NODES)
        def _():
            cpx.start()

        @pl.when(nbase + ROW_CHUNK == N_NODES + HC)
        def _():
            cpxh.start()

        cpa.wait()
        cph.wait()

        @pl.when(nbase + ROW_CHUNK <= N_NODES)
        def _():
            cpx.wait()

        @pl.when(nbase + ROW_CHUNK == N_NODES + HC)
        def _():
            cpxh.wait()

        def upd_body(r, _):
            dv = plsc.load_gather(
                dinv1d, [jnp.full((16,), ch * ROW_CHUNK + r, _I32)])
            if not last:
                av = half * dv * dv
                bv = half * dv
            else:
                av = half * dv
                bv = jnp.full((16,), half, _F32)
            for j in range(DH // 16):
                sj = abuf[r, pl.ds(j * 16, 16)] + hbuf[r, pl.ds(j * 16, 16)]
                o = av * sj + bv * xbuf[r, pl.ds(j * 16, 16)]
                if last:
                    o = jnp.maximum(o, jnp.float32(0.0))
                abuf[r, pl.ds(j * 16, 16)] = o
            return 0
        @pl.when(nbase < N_NODES)
        def _():
            lax.fori_loop(0, ROW_CHUNK, upd_body, 0)
            if not last:
                pltpu.sync_copy(abuf, hpr.at[pl.ds(gbase, ROW_CHUNK), :])
            else:
                pltpu.sync_copy(
                    abuf, outp.at[pl.ds(nbase, ROW_CHUNK), pl.ds(coff, DH)])
        if not last:
            pltpu.sync_copy(zbuf, agg.at[pl.ds(nbase, ROW_CHUNK), :])
        return 0

    def hop01(k, _):
        edge_loop()
        plsc.subcore_barrier()
        lax.fori_loop(0, N_ROW_CHUNKS, lambda i, cc: upd_chunk(i, False), 0)
        plsc.subcore_barrier()
        return 0
    lax.fori_loop(0, K_ITERS - 1, hop01, 0)

    edge_loop()
    plsc.subcore_barrier()
    lax.fori_loop(0, N_ROW_CHUNKS, lambda i, cc: upd_chunk(i, True), 0)


@jax.jit
def _sc_propagate(xp, srcr, dstr):
    mesh = plsc.VectorSubcoreMesh(core_axis_name="c", subcore_axis_name="s",
                                  num_cores=NC, num_subcores=NS)
    f = pl.kernel(
        _sc_body,
        out_type=(jax.ShapeDtypeStruct((NPAD, D_FEAT), _F32),
                  jax.ShapeDtypeStruct((NC * NPAD, DH), _F32)),
        mesh=mesh,
        scratch_types=[
            pltpu.MemorySpace.VMEM_SHARED((NPAD, DH), _F32),
            pltpu.MemorySpace.VMEM_SHARED((NPAD, 16), _F32),
            pltpu.VMEM((ECHUNK,), _I32),
            pltpu.VMEM((ECHUNK,), _I32),
            pltpu.VMEM((ECHUNK // 2,), _I32),
            pltpu.VMEM((ECHUNK // 2,), _I32),
            pltpu.VMEM((ECHUNK // 2,), _I32),
            pltpu.VMEM((ECHUNK // 2,), _I32),
            pltpu.VMEM((ECHUNK // 2, DH), _F32),
            pltpu.VMEM((ECHUNK // 2, DH), _F32),
            pltpu.VMEM((ROW_CHUNK, DH), _F32),
            pltpu.VMEM((ROW_CHUNK, DH), _F32),
            pltpu.VMEM((ROW_CHUNK, DH), _F32),
            pltpu.VMEM((ROW_CHUNK, DH), _F32),
            pltpu.VMEM((ROW_CHUNK, 16), _F32),
            pltpu.VMEM((ECHUNK, 16), _F32),
            pltpu.VMEM((ROW_CHUNK, 16), _F32),
            pltpu.VMEM((ROWS_PER_TILE,), _F32),
            pltpu.SemaphoreType.DMA,
            pltpu.SemaphoreType.DMA,
            pltpu.SemaphoreType.DMA,
        ],
        compiler_params=pltpu.CompilerParams(use_tc_tiling_on_sc=False,
                                             needs_layout_passes=False),
        name="appnp_sc_propagate",
    )
    return f(xp, srcr, dstr)


def _mlp_body(p_ref, w1_ref, b1_ref, w2_ref, b2_ref, emb_ref, log_ref):
    pb = p_ref[...]
    emb = lax.dot_general(pb, w1_ref[...], (((1,), (1,)), ((), ())),
                          preferred_element_type=_F32) + b1_ref[...]
    emb_ref[...] = emb
    r = jnp.maximum(emb, jnp.float32(0.0))
    log_ref[...] = lax.dot_general(r, w2_ref[...], (((1,), (1,)), ((), ())),
                                   preferred_element_type=_F32) + b2_ref[...]


def _mlp(p, W1, b1, W2, b2):
    BR = 1000
    grid = (N_NODES // BR,)
    return pl.pallas_call(
        _mlp_body,
        grid=grid,
        in_specs=[
            pl.BlockSpec((BR, D_FEAT), lambda i: (i, 0)),
            pl.BlockSpec((D_FEAT, D_FEAT), lambda i: (0, 0)),
            pl.BlockSpec((1, D_FEAT), lambda i: (0, 0)),
            pl.BlockSpec((40, D_FEAT), lambda i: (0, 0)),
            pl.BlockSpec((1, 40), lambda i: (0, 0)),
        ],
        out_specs=[
            pl.BlockSpec((BR, D_FEAT), lambda i: (i, 0)),
            pl.BlockSpec((BR, 40), lambda i: (i, 0)),
        ],
        out_shape=[
            jax.ShapeDtypeStruct((N_NODES, D_FEAT), _F32),
            jax.ShapeDtypeStruct((N_NODES, 40), _F32),
        ],
    )(p, W1, b1, W2, b2)


def kernel(x, edge_index, W1, b1, W2, b2):
    src = edge_index[0].astype(_I32)
    dst = edge_index[1].astype(_I32)
    npad_e = EPAD - E_EDGES
    pad_idx = (N_NODES + jnp.arange(npad_e, dtype=_I32) % (NPAD - N_NODES))
    srcr = jnp.concatenate([src, pad_idx])
    dstr = jnp.concatenate([dst, pad_idx])
    outp, _ = _sc_propagate(x, srcr, dstr)
    return _mlp(outp, W1, b1.reshape(1, -1), W2, b2.reshape(1, -1))


# speedup vs baseline: 1.2021x; 1.0169x over previous
import jax
import jax.numpy as jnp
from jax import lax
from jax.experimental import pallas as pl
from jax.experimental.pallas import tpu as pltpu
from jax.experimental.pallas import tpu_sc as plsc

N_NODES = 10000
K_ITERS = 3
ALPHA = 0.5
D_FEAT = 256
DH = 128
NC, NS = 2, 16
NPAD = 10240
ROWS_PER_TILE = NPAD // NS
ROW_CHUNK = 32
N_ROW_CHUNKS = ROWS_PER_TILE // ROW_CHUNK
E_EDGES = 160000
EPAD = 163840
EDGES_PER_TILE = EPAD // NS
ECHUNK = 128
N_ECHUNKS = EDGES_PER_TILE // ECHUNK

_F32 = jnp.float32
_I32 = jnp.int32


def _sc_body(xp, srcr, dstr, outp, hpr,
             agg, deg2d, sbuf, dbuf, sadjA, sadjB, dadjA, dadjB,
             gbufA, gbufB, abuf, hbuf, xbuf, zbuf, zb16, ones16, degbuf,
             dinv1d, semA, semB, semI, semST):
    c = lax.axis_index("c")
    s = lax.axis_index("s")
    half = jnp.float32(0.5)
    coff = c * DH

    zeros_v = jnp.zeros((16,), _F32)
    ones_v = jnp.ones((16,), _F32)

    def fill_body(r, _):
        ones16[r, pl.ds(0, 16)] = ones_v
        return 0
    lax.fori_loop(0, ECHUNK, fill_body, 0)

    def fillz_body(r, _):
        for j in range(DH // 16):
            zbuf[r, pl.ds(j * 16, 16)] = zeros_v
        zb16[r, pl.ds(0, 16)] = zeros_v
        return 0
    lax.fori_loop(0, ROW_CHUNK, fillz_body, 0)

    def zero_both(ch, _):
        nbase = s * ROWS_PER_TILE + ch * ROW_CHUNK
        pltpu.sync_copy(zbuf, agg.at[pl.ds(nbase, ROW_CHUNK), :])
        pltpu.sync_copy(zb16, deg2d.at[pl.ds(nbase, ROW_CHUNK), :])
        return 0
    lax.fori_loop(0, N_ROW_CHUNKS, zero_both, 0)
    plsc.subcore_barrier()

    pltpu.sync_copy(dstr.at[pl.ds(s * EDGES_PER_TILE, ECHUNK)], dbuf)

    def deg_pair(t, _):
        off = s * EDGES_PER_TILE + 2 * t * ECHUNK
        pltpu.async_copy(dstr.at[pl.ds(off + ECHUNK, ECHUNK)], sbuf, semA)
        pltpu.sync_copy(ones16, deg2d.at[dbuf], add=True)

        @pl.when(t < N_ECHUNKS // 2 - 1)
        def _():
            pltpu.async_copy(
                dstr.at[pl.ds(off + 2 * ECHUNK, ECHUNK)], dbuf, semB)

        pltpu.make_async_copy(dstr.at[pl.ds(0, ECHUNK)], sbuf, semA).wait()
        pltpu.sync_copy(ones16, deg2d.at[sbuf], add=True)

        @pl.when(t < N_ECHUNKS // 2 - 1)
        def _():
            pltpu.make_async_copy(
                dstr.at[pl.ds(0, ECHUNK)], dbuf, semB).wait()
        return 0
    lax.fori_loop(0, N_ECHUNKS // 2, deg_pair, 0)
    plsc.subcore_barrier()

    def newton_chunk(ch, _):
        nbase = s * ROWS_PER_TILE + ch * ROW_CHUNK
        pltpu.sync_copy(deg2d.at[pl.ds(nbase, ROW_CHUNK), :], degbuf)

        iota_v = lax.iota(_I32, 16)
        zero_i = jnp.zeros((16,), _I32)
        for t in range(ROW_CHUNK // 16):
            rows = jnp.full((16,), t * 16, _I32) + iota_v
            d = plsc.load_gather(degbuf, [rows, zero_i]) + ones_v
            sq = jnp.full((16,), 512.0, _F32)
            for thresh, seed in ((65536.0, 256.0), (16384.0, 128.0),
                                 (4096.0, 64.0), (1024.0, 32.0),
                                 (256.0, 16.0), (64.0, 8.0),
                                 (16.0, 4.0), (4.0, 2.0)):
                sq = jnp.where(d < jnp.float32(thresh),
                               jnp.full((16,), seed, _F32), sq)
            for _it in range(5):
                sq = (sq + d / sq) * half
            out_rows = jnp.full((16,), ch * ROW_CHUNK + t * 16, _I32) + iota_v
            plsc.store_scatter(dinv1d, [out_rows], ones_v / sq)
        return 0
    lax.fori_loop(0, N_ROW_CHUNKS, newton_chunk, 0)

    HC = ROW_CHUNK // 2

    def h0_chunk(ch, _):
        nbase = s * ROWS_PER_TILE + ch * ROW_CHUNK
        gbase = c * NPAD + nbase

        @pl.when(nbase + ROW_CHUNK <= N_NODES)
        def _():
            pltpu.sync_copy(
                xp.at[pl.ds(nbase, ROW_CHUNK), pl.ds(coff, DH)], xbuf)

        @pl.when(nbase + ROW_CHUNK == N_NODES + HC)
        def _():
            pltpu.sync_copy(xp.at[pl.ds(nbase, HC), pl.ds(coff, DH)],
                            xbuf.at[pl.ds(0, HC), :])

        @pl.when(jnp.logical_and(ch > 0, nbase - ROW_CHUNK < N_NODES))
        def _():
            pltpu.make_async_copy(
                hbuf, hpr.at[pl.ds(gbase, ROW_CHUNK), :], semST).wait()

        @pl.when(nbase < N_NODES)
        def _():
            def h0_body(r, _):
                dv = plsc.load_gather(
                    dinv1d, [jnp.full((16,), ch * ROW_CHUNK + r, _I32)])
                for j in range(DH // 16):
                    hbuf[r, pl.ds(j * 16, 16)] = (
                        xbuf[r, pl.ds(j * 16, 16)] * dv)
                return 0
            lax.fori_loop(0, ROW_CHUNK, h0_body, 0)
            pltpu.async_copy(hbuf, hpr.at[pl.ds(gbase, ROW_CHUNK), :], semST)
        return 0
    lax.fori_loop(0, N_ROW_CHUNKS, h0_chunk, 0)
    last_nb = s * ROWS_PER_TILE + (N_ROW_CHUNKS - 1) * ROW_CHUNK

    @pl.when(last_nb < N_NODES)
    def _():
        pltpu.make_async_copy(
            hbuf, hpr.at[pl.ds(0, ROW_CHUNK), :], semST).wait()
    plsc.subcore_barrier()

    coff_v = jnp.full((16,), c * NPAD, _I32)

    H = ECHUNK // 2

    def load_group(g):
        off = s * EDGES_PER_TILE + g * ECHUNK
        pltpu.sync_copy(srcr.at[pl.ds(off, ECHUNK)], sbuf)
        pltpu.sync_copy(dstr.at[pl.ds(off, ECHUNK)], dbuf)

    def build_half(sadj, dadj, base):
        for i in range(H // 16):
            sadj[pl.ds(i * 16, 16)] = sbuf[pl.ds(base + i * 16, 16)] + coff_v
            dadj[pl.ds(i * 16, 16)] = dbuf[pl.ds(base + i * 16, 16)]

    def edge_loop():
        load_group(0)
        build_half(sadjA, dadjA, 0)
        build_half(sadjB, dadjB, H)
        gA = pltpu.async_copy(hpr.at[sadjA], gbufA, semA)

        def edge_group(g, _):
            pltpu.async_copy(hpr.at[sadjB], gbufB, semB)

            @pl.when(g < N_ECHUNKS - 1)
            def _():
                off = s * EDGES_PER_TILE + (g + 1) * ECHUNK
                pltpu.async_copy(srcr.at[pl.ds(off, ECHUNK)], sbuf, semI)
                pltpu.async_copy(dstr.at[pl.ds(off, ECHUNK)], dbuf, semI)

            pltpu.make_async_copy(hpr.at[sadjA], gbufA, semA).wait()
            pltpu.sync_copy(gbufA, agg.at[dadjA], add=True)

            @pl.when(g < N_ECHUNKS - 1)
            def _():
                pltpu.make_async_copy(
                    srcr.at[pl.ds(0, ECHUNK)], sbuf, semI).wait()
                pltpu.make_async_copy(
                    dstr.at[pl.ds(0, ECHUNK)], dbuf, semI).wait()
                build_half(sadjA, dadjA, 0)
                pltpu.async_copy(hpr.at[sadjA], gbufA, semA)

            pltpu.make_async_copy(hpr.at[sadjB], gbufB, semB).wait()
            pltpu.sync_copy(gbufB, agg.at[dadjB], add=True)

            @pl.when(g < N_ECHUNKS - 1)
            def _():
                build_half(sadjB, dadjB, H)
            return 0
        lax.fori_loop(0, N_ECHUNKS, edge_group, 0)

    def upd_chunk(ch, last):
        nbase = s * ROWS_PER_TILE + ch * ROW_CHUNK
        gbase = c * NPAD + nbase
        @pl.when(jnp.logical_and(ch > 0, nbase - ROW_CHUNK < N_NODES))
        def _():
            pltpu.make_async_copy(
                abuf, hpr.at[pl.ds(gbase, ROW_CHUNK), :], semST).wait()

        cpa = pltpu.make_async_copy(agg.at[pl.ds(nbase, ROW_CHUNK), :],
                                    abuf, semA)
        cph = pltpu.make_async_copy(hpr.at[pl.ds(gbase, ROW_CHUNK), :],
                                    hbuf, semB)
        cpx = pltpu.make_async_copy(
            xp.at[pl.ds(nbase, ROW_CHUNK), pl.ds(coff, DH)], xbuf, semI)
        cpxh = pltpu.make_async_copy(
            xp.at[pl.ds(nbase, HC), pl.ds(coff, DH)],
            xbuf.at[pl.ds(0, HC), :], semI)
        cpa.start()
        cph.start()

        @pl.when(nbase + ROW_CHUNK <= N_NODES)
        def _():
            cpx.start()

        @pl.when(nbase + ROW_CHUNK == N_NODES + HC)
        def _():
            cpxh.start()

        cpa.wait()
        cph.wait()

        @pl.when(nbase + ROW_CHUNK <= N_NODES)
        def _():
            cpx.wait()

        @pl.when(nbase + ROW_CHUNK == N_NODES + HC)
        def _():
            cpxh.wait()

        def upd_body(r, _):
            dv = plsc.load_gather(
                dinv1d, [jnp.full((16,), ch * ROW_CHUNK + r, _I32)])
            if not last:
                av = half * dv * dv
                bv = half * dv
            else:
                av = half * dv
                bv = jnp.full((16,), half, _F32)
            for j in range(DH // 16):
                sj = abuf[r, pl.ds(j * 16, 16)] + hbuf[r, pl.ds(j * 16, 16)]
                o = av * sj + bv * xbuf[r, pl.ds(j * 16, 16)]
                if last:
                    o = jnp.maximum(o, jnp.float32(0.0))
                abuf[r, pl.ds(j * 16, 16)] = o
            return 0
        @pl.when(nbase < N_NODES)
        def _():
            lax.fori_loop(0, ROW_CHUNK, upd_body, 0)
            if not last:
                pltpu.async_copy(
                    abuf, hpr.at[pl.ds(gbase, ROW_CHUNK), :], semST)
            else:
                pltpu.async_copy(
                    abuf, outp.at[pl.ds(nbase, ROW_CHUNK), pl.ds(coff, DH)],
                    semST)
        if not last:
            pltpu.sync_copy(zbuf, agg.at[pl.ds(nbase, ROW_CHUNK), :])
        return 0

    def drain_upd(target):
        nb_last = s * ROWS_PER_TILE + (N_ROW_CHUNKS - 1) * ROW_CHUNK

        @pl.when(nb_last < N_NODES)
        def _():
            pltpu.make_async_copy(
                abuf, target.at[pl.ds(0, ROW_CHUNK), pl.ds(0, DH)],
                semST).wait()

    def hop01(k, _):
        edge_loop()
        plsc.subcore_barrier()
        lax.fori_loop(0, N_ROW_CHUNKS, lambda i, cc: upd_chunk(i, False), 0)
        drain_upd(hpr)
        plsc.subcore_barrier()
        return 0
    lax.fori_loop(0, K_ITERS - 1, hop01, 0)

    edge_loop()
    plsc.subcore_barrier()
    lax.fori_loop(0, N_ROW_CHUNKS, lambda i, cc: upd_chunk(i, True), 0)
    drain_upd(outp)


@jax.jit
def _sc_propagate(xp, srcr, dstr):
    mesh = plsc.VectorSubcoreMesh(core_axis_name="c", subcore_axis_name="s",
                                  num_cores=NC, num_subcores=NS)
    f = pl.kernel(
        _sc_body,
        out_type=(jax.ShapeDtypeStruct((NPAD, D_FEAT), _F32),
                  jax.ShapeDtypeStruct((NC * NPAD, DH), _F32)),
        mesh=mesh,
        scratch_types=[
            pltpu.MemorySpace.VMEM_SHARED((NPAD, DH), _F32),
            pltpu.MemorySpace.VMEM_SHARED((NPAD, 16), _F32),
            pltpu.VMEM((ECHUNK,), _I32),
            pltpu.VMEM((ECHUNK,), _I32),
            pltpu.VMEM((ECHUNK // 2,), _I32),
            pltpu.VMEM((ECHUNK // 2,), _I32),
            pltpu.VMEM((ECHUNK // 2,), _I32),
            pltpu.VMEM((ECHUNK // 2,), _I32),
            pltpu.VMEM((ECHUNK // 2, DH), _F32),
            pltpu.VMEM((ECHUNK // 2, DH), _F32),
            pltpu.VMEM((ROW_CHUNK, DH), _F32),
            pltpu.VMEM((ROW_CHUNK, DH), _F32),
            pltpu.VMEM((ROW_CHUNK, DH), _F32),
            pltpu.VMEM((ROW_CHUNK, DH), _F32),
            pltpu.VMEM((ROW_CHUNK, 16), _F32),
            pltpu.VMEM((ECHUNK, 16), _F32),
            pltpu.VMEM((ROW_CHUNK, 16), _F32),
            pltpu.VMEM((ROWS_PER_TILE,), _F32),
            pltpu.SemaphoreType.DMA,
            pltpu.SemaphoreType.DMA,
            pltpu.SemaphoreType.DMA,
            pltpu.SemaphoreType.DMA,
        ],
        compiler_params=pltpu.CompilerParams(use_tc_tiling_on_sc=False,
                                             needs_layout_passes=False),
        name="appnp_sc_propagate",
    )
    return f(xp, srcr, dstr)


def _mlp_body(p_ref, w1_ref, b1_ref, w2_ref, b2_ref, emb_ref, log_ref):
    pb = p_ref[...]
    emb = lax.dot_general(pb, w1_ref[...], (((1,), (1,)), ((), ())),
                          preferred_element_type=_F32) + b1_ref[...]
    emb_ref[...] = emb
    r = jnp.maximum(emb, jnp.float32(0.0))
    log_ref[...] = lax.dot_general(r, w2_ref[...], (((1,), (1,)), ((), ())),
                                   preferred_element_type=_F32) + b2_ref[...]


def _mlp(p, W1, b1, W2, b2):
    BR = 1000
    grid = (N_NODES // BR,)
    return pl.pallas_call(
        _mlp_body,
        grid=grid,
        in_specs=[
            pl.BlockSpec((BR, D_FEAT), lambda i: (i, 0)),
            pl.BlockSpec((D_FEAT, D_FEAT), lambda i: (0, 0)),
            pl.BlockSpec((1, D_FEAT), lambda i: (0, 0)),
            pl.BlockSpec((40, D_FEAT), lambda i: (0, 0)),
            pl.BlockSpec((1, 40), lambda i: (0, 0)),
        ],
        out_specs=[
            pl.BlockSpec((BR, D_FEAT), lambda i: (i, 0)),
            pl.BlockSpec((BR, 40), lambda i: (i, 0)),
        ],
        out_shape=[
            jax.ShapeDtypeStruct((N_NODES, D_FEAT), _F32),
            jax.ShapeDtypeStruct((N_NODES, 40), _F32),
        ],
    )(p, W1, b1, W2, b2)


def kernel(x, edge_index, W1, b1, W2, b2):
    src = edge_index[0].astype(_I32)
    dst = edge_index[1].astype(_I32)
    npad_e = EPAD - E_EDGES
    pad_idx = (N_NODES + jnp.arange(npad_e, dtype=_I32) % (NPAD - N_NODES))
    srcr = jnp.concatenate([src, pad_idx])
    dstr = jnp.concatenate([dst, pad_idx])
    outp, _ = _sc_propagate(x, srcr, dstr)
    return _mlp(outp, W1, b1.reshape(1, -1), W2, b2.reshape(1, -1))
